# baseline (speedup 1.0000x reference)
import functools

import jax
import jax.numpy as jnp
from jax import lax
from jax.experimental import pallas as pl
from jax.experimental.pallas import tpu as pltpu
from jax.experimental.pallas import tpu_sc as plsc

D_MODEL = 32
CHUNK = 128
GDEPTH = 6
WDEPTH = 3

_kernel_cache = {}


def _build(n_s, n_bt, nc, ns):
    num_chunks = n_s * n_bt
    nw = nc * ns
    chunks_per_w = num_chunks // nw
    mesh = plsc.VectorSubcoreMesh(
        core_axis_name="c", subcore_axis_name="s", num_cores=nc, num_subcores=ns
    )

    @functools.partial(
        pl.kernel,
        mesh=mesh,
        compiler_params=pltpu.CompilerParams(
            use_tc_tiling_on_sc=False, needs_layout_passes=False
        ),
        out_type=jax.ShapeDtypeStruct((n_s, 4, n_bt, 8 * CHUNK), jnp.float32),
        scratch_types=[
            pltpu.VMEM((chunks_per_w, CHUNK), jnp.int32),
            pltpu.VMEM((GDEPTH * CHUNK, D_MODEL), jnp.float32),
            pltpu.VMEM((CHUNK * (D_MODEL + 1),), jnp.float32),
            pltpu.VMEM((WDEPTH, 4, 1, 8 * CHUNK), jnp.float32),
            pltpu.SemaphoreType.DMA,
            pltpu.SemaphoreType.DMA,
        ],
    )
    def gather_kernel(
        table_hbm, idx_hbm, out_hbm, idx_v, rows_v, pad_v, t_v, gsem, wsem
    ):
        wid = lax.axis_index("s") * nc + lax.axis_index("c")
        base = wid * chunks_per_w
        pltpu.sync_copy(idx_hbm.at[pl.ds(base, chunks_per_w)], idx_v)

        iota16 = lax.iota(jnp.int32, 16)

        def fire_gather(j):
            slot = lax.rem(j, GDEPTH)
            pltpu.async_copy(
                table_hbm.at[idx_v.at[j]],
                rows_v.at[pl.ds(slot * CHUNK, CHUNK)],
                gsem,
            )

        def drain_gather(j):
            slot = lax.rem(j, GDEPTH)
            pltpu.make_async_copy(
                table_hbm.at[idx_v.at[j]],
                rows_v.at[pl.ds(slot * CHUNK, CHUNK)],
                gsem,
            ).wait()

        def fire_write(j, tb):
            t = base + j
            s = lax.div(t, n_bt)
            bt = lax.rem(t, n_bt)
            pltpu.async_copy(
                t_v.at[pl.ds(tb, 1)],
                out_hbm.at[pl.ds(s, 1), pl.ds(0, 4), pl.ds(bt, 1)],
                wsem,
            )

        def drain_write(tb):
            pltpu.make_async_copy(
                table_hbm.at[pl.ds(0, 8 * CHUNK // D_MODEL)],
                t_v.at[pl.ds(tb, 1)],
                wsem,
            ).wait()

        P = D_MODEL + 1
        iota_p = iota16 * P

        def transpose_chunk(j, tb):
            slot = lax.rem(j, GDEPTH)
            rbase = slot * CHUNK
            for g in range(CHUNK // 16):
                for i in range(16):
                    r = 16 * g + i
                    pad_v[pl.ds(r * P, 16)] = rows_v[rbase + r, pl.ds(0, 16)]
                    pad_v[pl.ds(r * P + 16, 16)] = rows_v[rbase + r, pl.ds(16, 16)]
                rvec = iota_p + (16 * g * P)
                for d in range(D_MODEL):
                    v = plsc.load_gather(pad_v, [rvec + d])
                    t_v[tb, d // 8, 0, pl.ds((d % 8) * CHUNK + 16 * g, 16)] = v

        for j in range(GDEPTH - 1):
            fire_gather(j)

        def body(j, carry):
            tb = lax.rem(j, WDEPTH)
            drain_gather(j)

            @pl.when(j + GDEPTH - 1 < chunks_per_w)
            def _():
                fire_gather(j + GDEPTH - 1)

            @pl.when(j >= WDEPTH)
            def _():
                drain_write(tb)

            transpose_chunk(j, tb)
            fire_write(j, tb)
            return carry

        lax.fori_loop(0, chunks_per_w, body, 0)
        for k in range(WDEPTH):
            drain_write(lax.rem(chunks_per_w - WDEPTH + k, WDEPTH))

    return gather_kernel


def kernel(x, table):
    n_b, n_s = x.shape
    n_bt = n_b // CHUNK
    key = (n_s, n_bt)
    if key not in _kernel_cache:
        info = plsc.get_sparse_core_info()
        _kernel_cache[key] = _build(n_s, n_bt, info.num_cores, info.num_subcores)
    xt = jnp.transpose(x).reshape(n_s * n_bt, CHUNK)
    t5 = _kernel_cache[key](table, xt)
    out = (
        t5.reshape(n_s, 4, n_bt, 8, CHUNK)
        .transpose(0, 1, 3, 2, 4)
        .reshape(n_s, D_MODEL, n_b)
        .transpose(2, 0, 1)
    )
    return out


# speedup vs baseline: 1.0026x; 1.0026x over previous
import functools

import jax
import jax.numpy as jnp
from jax import lax
from jax.experimental import pallas as pl
from jax.experimental.pallas import tpu as pltpu
from jax.experimental.pallas import tpu_sc as plsc

D_MODEL = 32
CHUNK = 128
GDEPTH = 4
WDEPTH = 2

_kernel_cache = {}


def _build(n_s, n_bt, nc, ns):
    num_chunks = n_s * n_bt
    nw = nc * ns
    chunks_per_w = num_chunks // nw
    mesh = plsc.VectorSubcoreMesh(
        core_axis_name="c", subcore_axis_name="s", num_cores=nc, num_subcores=ns
    )

    @functools.partial(
        pl.kernel,
        mesh=mesh,
        compiler_params=pltpu.CompilerParams(
            use_tc_tiling_on_sc=False, needs_layout_passes=False
        ),
        out_type=jax.ShapeDtypeStruct((n_s, 4, n_bt, 8 * CHUNK), jnp.float32),
        scratch_types=[
            pltpu.VMEM((chunks_per_w, CHUNK), jnp.int32),
            pltpu.VMEM((GDEPTH * CHUNK, D_MODEL), jnp.float32),
            pltpu.VMEM((CHUNK * (D_MODEL + 1),), jnp.float32),
            pltpu.VMEM((WDEPTH, 4, 1, 8 * CHUNK), jnp.float32),
            pltpu.SemaphoreType.DMA,
            pltpu.SemaphoreType.DMA,
        ],
    )
    def gather_kernel(
        table_hbm, idx_hbm, out_hbm, idx_v, rows_v, pad_v, t_v, gsem, wsem
    ):
        wid = lax.axis_index("s") * nc + lax.axis_index("c")
        base = wid * chunks_per_w
        pltpu.sync_copy(idx_hbm.at[pl.ds(base, chunks_per_w)], idx_v)

        iota16 = lax.iota(jnp.int32, 16)

        def fire_gather(j):
            slot = lax.rem(j, GDEPTH)
            pltpu.async_copy(
                table_hbm.at[idx_v.at[j]],
                rows_v.at[pl.ds(slot * CHUNK, CHUNK)],
                gsem,
            )

        def drain_gather(j):
            slot = lax.rem(j, GDEPTH)
            pltpu.make_async_copy(
                table_hbm.at[idx_v.at[j]],
                rows_v.at[pl.ds(slot * CHUNK, CHUNK)],
                gsem,
            ).wait()

        def fire_write(j, tb):
            t = base + j
            s = lax.div(t, n_bt)
            bt = lax.rem(t, n_bt)
            pltpu.async_copy(
                t_v.at[pl.ds(tb, 1)],
                out_hbm.at[pl.ds(s, 1), pl.ds(0, 4), pl.ds(bt, 1)],
                wsem,
            )

        def drain_write(tb):
            pltpu.make_async_copy(
                table_hbm.at[pl.ds(0, 8 * CHUNK // D_MODEL)],
                t_v.at[pl.ds(tb, 1)],
                wsem,
            ).wait()

        P = D_MODEL + 1
        iota_p = iota16 * P

        def transpose_chunk(j, tb):
            slot = lax.rem(j, GDEPTH)
            rbase = slot * CHUNK
            for i in range(CHUNK):
                pad_v[pl.ds(i * P, 16)] = rows_v[rbase + i, pl.ds(0, 16)]
                pad_v[pl.ds(i * P + 16, 16)] = rows_v[rbase + i, pl.ds(16, 16)]
            for g in range(CHUNK // 16):
                rvec = iota_p + (16 * g * P)
                for d in range(D_MODEL):
                    v = plsc.load_gather(pad_v, [rvec + d])
                    t_v[tb, d // 8, 0, pl.ds((d % 8) * CHUNK + 16 * g, 16)] = v

        for j in range(GDEPTH):
            fire_gather(j)

        def body(j, carry):
            tb = lax.rem(j, WDEPTH)
            drain_gather(j)

            @pl.when(j >= WDEPTH)
            def _():
                drain_write(tb)

            transpose_chunk(j, tb)
            fire_write(j, tb)

            @pl.when(j + GDEPTH < chunks_per_w)
            def _():
                fire_gather(j + GDEPTH)

            return carry

        lax.fori_loop(0, chunks_per_w, body, 0)
        for k in range(WDEPTH):
            drain_write(lax.rem(chunks_per_w - WDEPTH + k, WDEPTH))

    return gather_kernel


def kernel(x, table):
    n_b, n_s = x.shape
    n_bt = n_b // CHUNK
    key = (n_s, n_bt)
    if key not in _kernel_cache:
        info = plsc.get_sparse_core_info()
        _kernel_cache[key] = _build(n_s, n_bt, info.num_cores, info.num_subcores)
    xt = jnp.transpose(x).reshape(n_s * n_bt, CHUNK)
    t5 = _kernel_cache[key](table, xt)
    out = (
        t5.reshape(n_s, 4, n_bt, 8, CHUNK)
        .transpose(0, 1, 3, 2, 4)
        .reshape(n_s, D_MODEL, n_b)
        .transpose(2, 0, 1)
    )
    return out
